# baseline (speedup 1.0000x reference)
import functools

import jax
import jax.numpy as jnp
from jax import lax
from jax.experimental import pallas as pl
from jax.experimental.pallas import tpu as pltpu
from jax.experimental.pallas import tpu_sc as plsc

N = 10000
E = 320000
D = 128
G = 64

NC = 2
NS = 16
NW = NC * NS
EPT = E // NW
CH = 80
NCH = EPT // CH
NPAD = 10112
RPS = NPAD // NS

_SELU_ALPHA = 1.6732632423543772848170429916717
_SELU_SCALE = 1.0507009873554804934193349852946


def _selu(x):
    safe = jnp.minimum(x, 0.0)
    return _SELU_SCALE * jnp.where(
        x > 0, x, _SELU_ALPHA * (jnp.exp(safe) - 1.0))


def _sc_layer_body(src_h, dst_h, asrc_h, adst_h, h_h, z2_h, z1_h,
                   acc_o, den_o, *sc):
    src_c = list(sc[0:6])
    dst_c = list(sc[6:12])
    dst_s = list(sc[12:15])
    as_c = list(sc[15:19])
    ad_c = list(sc[19:23])
    w_c = list(sc[23:26])
    rows = list(sc[26:29])
    den_b = sc[29]
    acc_sh = sc[30]
    den_sh = sc[31]
    g1s = list(sc[32:38])
    g2s = list(sc[38:42])
    hs = list(sc[42:45])
    ss = list(sc[45:48])

    cid = lax.axis_index("c")
    sid = lax.axis_index("s")
    wid = cid * NS + sid
    r0 = sid * RPS

    pltpu.sync_copy(z2_h.at[pl.ds(r0, RPS)], acc_sh.at[pl.ds(r0, RPS)])
    pltpu.sync_copy(z1_h.at[pl.ds(r0, RPS)], den_b)
    pltpu.sync_copy(den_b, den_sh.at[pl.ds(r0, RPS)])
    plsc.subcore_barrier()

    def g1_descs(c, s6):
        o = (wid * NCH + c) * CH
        return (pltpu.make_async_copy(src_h.at[pl.ds(o, CH)], src_c[s6],
                                      g1s[s6]),
                pltpu.make_async_copy(dst_h.at[pl.ds(o, CH)], dst_c[s6],
                                      g1s[s6]))

    def a_descs(sa, s6):
        return (pltpu.make_async_copy(asrc_h.at[src_c[s6]], as_c[sa], g2s[sa]),
                pltpu.make_async_copy(adst_h.at[dst_c[s6]], ad_c[sa], g2s[sa]))

    def h_descs(sr, s6):
        return (pltpu.make_async_copy(h_h.at[src_c[s6]], rows[sr], hs[sr]),)

    def s_descs(sr):
        return (pltpu.make_async_copy(rows[sr], acc_sh.at[dst_s[sr]], ss[sr]),
                pltpu.make_async_copy(w_c[sr], den_sh.at[dst_s[sr]], ss[sr]))

    def issue_g1(c, s6):
        o = (wid * NCH + c) * CH
        pltpu.async_copy(src_h.at[pl.ds(o, CH)], src_c[s6], g1s[s6])
        pltpu.async_copy(dst_h.at[pl.ds(o, CH)], dst_c[s6], g1s[s6])

    def issue_a(sa, s6):
        pltpu.async_copy(asrc_h.at[src_c[s6]], as_c[sa], g2s[sa])
        pltpu.async_copy(adst_h.at[dst_c[s6]], ad_c[sa], g2s[sa])

    def issue_h(sr, s6):
        pltpu.async_copy(h_h.at[src_c[s6]], rows[sr], hs[sr])

    def issue_s(sr):
        pltpu.async_copy(rows[sr], acc_sh.at[dst_s[sr]], ss[sr], add=True)
        pltpu.async_copy(w_c[sr], den_sh.at[dst_s[sr]], ss[sr], add=True)

    def wait_all(descs):
        for d in descs:
            d.wait()

    def process(sr, sa, s6):
        for j in range(CH // 16):
            dst_s[sr][pl.ds(j * 16, 16)] = dst_c[s6][pl.ds(j * 16, 16)]
            a = as_c[sa][pl.ds(j * 16, 16)] + ad_c[sa][pl.ds(j * 16, 16)]
            a = jnp.maximum(a, 0.2 * a)
            w_c[sr][pl.ds(j * 16, 16)] = jnp.exp(a)

        @plsc.parallel_loop(0, CH, step=4, unroll=5)
        def _scale(e4):
            for i in range(4):
                e = e4 + i
                wb = plsc.load_gather(w_c[sr],
                                      [jnp.full((16,), e, jnp.int32)])
                for j in range(D // 16):
                    rows[sr][e, pl.ds(j * 16, 16)] = (
                        rows[sr][e, pl.ds(j * 16, 16)] * wb)

    def step(c, k, dynamic):
        s3, s3n = k % 3, (k + 1) % 3
        s6, s6_1 = k % 6, (k + 1) % 6
        s6_2, s6_4 = (k + 2) % 6, (k + 4) % 6
        sa, sa2 = k % 4, (k + 2) % 4

        if dynamic:
            @pl.when(c >= 2)
            def _():
                wait_all(s_descs(s3n))
        else:
            wait_all(s_descs(s3n))
        if (not dynamic and c + 2 <= NCH - 1) or dynamic:
            wait_all(g1_descs(c + 2, s6_2))
            issue_a(sa2, s6_2)
        if (not dynamic and c + 4 <= NCH - 1) or dynamic:
            issue_g1(c + 4, s6_4)
        if (not dynamic and c + 1 <= NCH - 1) or dynamic:
            issue_h(s3n, s6_1)
        wait_all(a_descs(sa, s6))
        wait_all(h_descs(s3, s6))
        process(s3, sa, s6)
        issue_s(s3)

    for c0 in range(4):
        issue_g1(c0, c0)
    wait_all(g1_descs(0, 0))
    issue_a(0, 0)
    issue_h(0, 0)
    wait_all(g1_descs(1, 1))
    issue_a(1, 1)

    def cbody(cc, carry):
        for k in range(12):
            step(cc * 12 + k, k, True)
        return carry

    lax.fori_loop(0, 10, cbody, 0)

    for c in range(120, NCH):
        step(c, c % 12, False)
    wait_all(s_descs((NCH - 2) % 3))
    wait_all(s_descs((NCH - 1) % 3))
    plsc.subcore_barrier()

    pltpu.sync_copy(acc_sh.at[pl.ds(r0, RPS)], acc_o.at[cid, pl.ds(r0, RPS)])
    pltpu.sync_copy(den_sh.at[pl.ds(r0, RPS)], den_b)
    pltpu.sync_copy(den_b, den_o.at[pl.ds(cid * NPAD + r0, RPS)])


_sc_layer = pl.kernel(
    _sc_layer_body,
    out_type=[
        jax.ShapeDtypeStruct((NC, NPAD, D), jnp.float32),
        jax.ShapeDtypeStruct((NC * NPAD,), jnp.float32),
    ],
    mesh=plsc.VectorSubcoreMesh(core_axis_name="c", subcore_axis_name="s"),
    compiler_params=pltpu.CompilerParams(needs_layout_passes=False),
    scratch_types=(
        [pltpu.VMEM((CH,), jnp.int32) for _ in range(15)]
        + [pltpu.VMEM((CH,), jnp.float32) for _ in range(11)]
        + [pltpu.VMEM((CH, D), jnp.float32) for _ in range(3)]
        + [
            pltpu.VMEM((RPS,), jnp.float32),
            pltpu.VMEM_SHARED((NPAD, D), jnp.float32),
            pltpu.VMEM_SHARED((NPAD,), jnp.float32),
        ]
        + [pltpu.SemaphoreType.DMA for _ in range(16)]
    ),
)


def _tc_pre_body(u_ref, w_ref, asw_ref, adw_ref, h_ref, asrc_ref, adst_ref,
                 wself_ref):
    h = jnp.dot(u_ref[...], w_ref[...], preferred_element_type=jnp.float32)
    h_ref[...] = h
    asrc = jnp.dot(h, asw_ref[...], preferred_element_type=jnp.float32)
    adst = jnp.dot(h, adw_ref[...], preferred_element_type=jnp.float32)
    asrc_ref[...] = asrc
    adst_ref[...] = adst
    a = asrc + adst
    wself_ref[...] = jnp.exp(jnp.maximum(a, 0.2 * a))


_tc_pre = pl.pallas_call(
    _tc_pre_body,
    out_shape=[
        jax.ShapeDtypeStruct((N, D), jnp.float32),
        jax.ShapeDtypeStruct((N, 1), jnp.float32),
        jax.ShapeDtypeStruct((N, 1), jnp.float32),
        jax.ShapeDtypeStruct((N, 1), jnp.float32),
    ],
)


def _tc_mid_body(acc_ref, den_ref, h_ref, wself_ref, b_ref, w_ref, asw_ref,
                 adw_ref, h2_ref, asrc_ref, adst_ref, wself2_ref):
    num = acc_ref[0, :N, :] + acc_ref[1, :N, :] + wself_ref[...] * h_ref[...]
    den = (den_ref[0:1, :N] + den_ref[1:2, :N]).reshape(N, 1) + \
        wself_ref[...] + 1e-16
    x = _selu(num / den + b_ref[...])
    h2 = jnp.dot(x, w_ref[...], preferred_element_type=jnp.float32)
    h2_ref[...] = h2
    asrc = jnp.dot(h2, asw_ref[...], preferred_element_type=jnp.float32)
    adst = jnp.dot(h2, adw_ref[...], preferred_element_type=jnp.float32)
    asrc_ref[...] = asrc
    adst_ref[...] = adst
    a = asrc + adst
    wself2_ref[...] = jnp.exp(jnp.maximum(a, 0.2 * a))


_tc_mid = pl.pallas_call(
    _tc_mid_body,
    out_shape=[
        jax.ShapeDtypeStruct((N, D), jnp.float32),
        jax.ShapeDtypeStruct((N, 1), jnp.float32),
        jax.ShapeDtypeStruct((N, 1), jnp.float32),
        jax.ShapeDtypeStruct((N, 1), jnp.float32),
    ],
)


def _tc_post_body(acc_ref, den_ref, h_ref, wself_ref, b_ref, batch_ref,
                  out_ref):
    num = acc_ref[0, :N, :] + acc_ref[1, :N, :] + wself_ref[...] * h_ref[...]
    den = (den_ref[0:1, :N] + den_ref[1:2, :N]).reshape(N, 1) + \
        wself_ref[...] + 1e-16
    y = _selu(num / den + b_ref[...])
    gids = lax.broadcasted_iota(jnp.int32, (G, N), 0)
    onehot = (gids == batch_ref[...]).astype(jnp.float32)
    sums = jnp.dot(onehot, y, preferred_element_type=jnp.float32)
    cnts = jnp.sum(onehot, axis=1, keepdims=True)
    out_ref[...] = sums / jnp.clip(cnts, 1.0, None)


_tc_post = pl.pallas_call(
    _tc_post_body,
    out_shape=jax.ShapeDtypeStruct((G, D), jnp.float32),
)


def kernel(u, edge_index, batch, W1, att_src1, att_dst1, b1,
           W2, att_src2, att_dst2, b2):
    ei = jnp.asarray(edge_index, jnp.int32)
    src3 = ei[0]
    dst3 = ei[1]
    batch2 = jnp.asarray(batch, jnp.int32).reshape(1, N)
    z2 = jnp.zeros((NPAD, D), jnp.float32)
    z1 = jnp.zeros((NPAD,), jnp.float32)

    h1, asrc1, adst1, wself1 = _tc_pre(
        u, W1, att_src1.reshape(D, 1), att_dst1.reshape(D, 1))
    acc1, den1 = _sc_layer(src3, dst3, asrc1.reshape(N), adst1.reshape(N),
                           h1, z2, z1)
    h2, asrc2, adst2, wself2 = _tc_mid(
        acc1, den1.reshape(NC, NPAD), h1, wself1, b1.reshape(1, D), W2,
        att_src2.reshape(D, 1), att_dst2.reshape(D, 1))
    acc2, den2 = _sc_layer(src3, dst3, asrc2.reshape(N), adst2.reshape(N),
                           h2, z2, z1)
    return _tc_post(acc2, den2.reshape(NC, NPAD), h2, wself2,
                    b2.reshape(1, D), batch2)
